# baseline (speedup 1.0000x reference)
import functools

import jax
import jax.numpy as jnp
from jax import lax
from jax.experimental import pallas as pl
from jax.experimental.pallas import tpu as pltpu
from jax.experimental.pallas import tpu_sc as plsc

B, N, K, EMB = 8, 1024, 20, 512
EPS = 1e-5


def _knn_body(ft_ref, xx_ref, idx_ref):
    b = pl.program_id(0)
    f = ft_ref[0]
    inner = lax.dot_general(f, f, (((1,), (1,)), ((), ())),
                            preferred_element_type=jnp.float32)
    lane = lax.broadcasted_iota(jnp.int32, (N, N), 1)
    sub = lax.broadcasted_iota(jnp.int32, (N, N), 0)
    eye = jnp.where(lane == sub, 1.0, 0.0)
    r_row = xx_ref[0]
    r_col = lax.dot_general(eye, r_row, (((1,), (1,)), ((), ())),
                            preferred_element_type=jnp.float32,
                            precision=lax.Precision.HIGHEST)
    cur = -r_row - (-2.0 * inner) - r_col
    base = b * N
    cols = []
    for _ in range(K):
        m = jnp.max(cur, axis=1, keepdims=True)
        cand = jnp.where(cur == m, lane, N)
        amin = jnp.min(cand, axis=1, keepdims=True)
        cols.append(amin + base)
        cur = jnp.where(lane == amin, -jnp.inf, cur)
    idx_ref[0] = jnp.concatenate(cols, axis=1)


def _knn(ft, xx, interpret=False):
    c = ft.shape[-1]
    return pl.pallas_call(
        _knn_body,
        grid=(B,),
        in_specs=[
            pl.BlockSpec((1, N, c), lambda b: (b, 0, 0)),
            pl.BlockSpec((1, 1, N), lambda b: (b, 0, 0)),
        ],
        out_specs=pl.BlockSpec((1, N, K), lambda b: (b, 0, 0)),
        out_shape=jax.ShapeDtypeStruct((B, N, K), jnp.int32),
        interpret=interpret,
    )(ft, xx)


_NW = 32
_RPW = (B * N) // _NW
_P = 8


def _make_sc_rel(cp):
    g2 = (_P * K) // 2
    nchunk = _RPW // _P
    mesh = plsc.VectorSubcoreMesh(core_axis_name="c", subcore_axis_name="s")

    @functools.partial(
        pl.kernel,
        out_type=jax.ShapeDtypeStruct((B * N * K, cp), jnp.float32),
        mesh=mesh,
        compiler_params=pltpu.CompilerParams(use_tc_tiling_on_sc=False),
        scratch_types=[
            pltpu.VMEM((2, g2), jnp.int32),
            pltpu.VMEM((2, g2), jnp.int32),
            pltpu.VMEM((2, _P * K, cp), jnp.float32),
            pltpu.VMEM((2, _P, cp), jnp.float32),
            pltpu.SemaphoreType.DMA,
            pltpu.SemaphoreType.DMA,
            pltpu.SemaphoreType.DMA,
            pltpu.SemaphoreType.DMA,
        ],
    )
    def sc_rel(ft_hbm, idx_hbm, rel_hbm, idx_a, idx_b, rows_v, fn_v,
               gsem0, gsem1, wsem0, wsem1):
        wid = lax.axis_index("s") * 2 + lax.axis_index("c")
        row0 = wid * _RPW

        def gathers(j, slot, gsem):
            r0 = row0 + j * _P
            e0 = r0 * K
            c1 = pltpu.make_async_copy(
                ft_hbm.at[idx_a.at[slot]],
                rows_v.at[slot, pl.ds(0, g2), :], gsem)
            c2 = pltpu.make_async_copy(
                ft_hbm.at[idx_b.at[slot]],
                rows_v.at[slot, pl.ds(g2, g2), :], gsem)
            return c1, c2

        def wcopy(j, slot, wsem):
            e0 = (row0 + j * _P) * K
            return pltpu.make_async_copy(
                rows_v.at[slot], rel_hbm.at[pl.ds(e0, _P * K)], wsem)

        def issue(j, slot, gsem):
            r0 = row0 + j * _P
            e0 = r0 * K
            pltpu.sync_copy(idx_hbm.at[pl.ds(e0, g2)], idx_a.at[slot])
            pltpu.sync_copy(idx_hbm.at[pl.ds(e0 + g2, g2)], idx_b.at[slot])
            c1, c2 = gathers(j, slot, gsem)
            c1.start()
            c2.start()
            pltpu.sync_copy(ft_hbm.at[pl.ds(r0, _P)], fn_v.at[slot])

        def process(j, slot, gsem, wsem):
            c1, c2 = gathers(j, slot, gsem)
            c1.wait()
            c2.wait()

            def point(p, carry2):
                def kgrp(k, carry3):
                    r = p * K + k
                    for ci in range(cp // 16):
                        c0 = ci * 16
                        rows_v[slot, r, pl.ds(c0, 16)] = (
                            rows_v[slot, r, pl.ds(c0, 16)]
                            - fn_v[slot, p, pl.ds(c0, 16)])
                    return carry3
                return lax.fori_loop(0, K, kgrp, carry2)

            lax.fori_loop(0, _P, point, 0)
            wcopy(j, slot, wsem).start()

        issue(0, 0, gsem0)

        def chunk(j, carry):
            @pl.when(lax.rem(j, 2) == 0)
            def _():
                @pl.when(j + 1 < nchunk)
                def _():
                    @pl.when(j >= 1)
                    def _():
                        wcopy(j - 1, 1, wsem1).wait()
                    issue(j + 1, 1, gsem1)
                process(j, 0, gsem0, wsem0)

            @pl.when(lax.rem(j, 2) == 1)
            def _():
                @pl.when(j + 1 < nchunk)
                def _():
                    wcopy(j - 1, 0, wsem0).wait()
                    issue(j + 1, 0, gsem0)
                process(j, 1, gsem1, wsem1)

            return carry

        lax.fori_loop(0, nchunk, chunk, 0)
        wcopy(nchunk - 2, 0, wsem0).wait()
        wcopy(nchunk - 1, 1, wsem1).wait()

    return sc_rel


_NB = 512


def _edgeconv_body(rel_ref, ft_ref, w_ref, mx_ref, mn_ref, s1_ref, s2_ref):
    w = w_ref[...]
    f = ft_ref[0]

    def yk(k):
        e = jnp.concatenate([rel_ref[0, :, k, :], f], axis=1)
        return lax.dot_general(e, w, (((1,), (0,)), ((), ())),
                               preferred_element_type=jnp.float32)

    y0 = yk(0)
    mx = y0
    mn = y0
    s1 = y0
    s2 = y0 * y0
    for k in range(1, K):
        y = yk(k)
        mx = jnp.maximum(mx, y)
        mn = jnp.minimum(mn, y)
        s1 = s1 + y
        s2 = s2 + y * y
    mx_ref[0] = mx
    mn_ref[0] = mn
    s1_ref[0] = s1
    s2_ref[0] = s2


def _edgeconv(rel, ft, w2c, interpret=False):
    cp = ft.shape[-1]
    o = w2c.shape[-1]
    out = functools.partial(
        pl.BlockSpec, (1, _NB, o), lambda b, i: (b, i, 0))
    res = pl.pallas_call(
        _edgeconv_body,
        grid=(B, N // _NB),
        in_specs=[
            pl.BlockSpec((1, _NB, K, cp), lambda b, i: (b, i, 0, 0)),
            pl.BlockSpec((1, _NB, cp), lambda b, i: (b, i, 0)),
            pl.BlockSpec((2 * cp, o), lambda b, i: (0, 0)),
        ],
        out_specs=[out() for _ in range(4)],
        out_shape=[jax.ShapeDtypeStruct((B, N, o), jnp.float32)
                   for _ in range(4)],
        interpret=interpret,
    )(rel.reshape(B, N, K, cp), ft, w2c)
    return [r.reshape(B * N, o) for r in res]


def _statsepi_body(mx_ref, mn_ref, s1_ref, s2_ref, g_ref, b_ref, out_ref):
    cnt = float(B * N * K)
    mean = jnp.sum(s1_ref[...], axis=0, keepdims=True) / cnt
    var = jnp.sum(s2_ref[...], axis=0, keepdims=True) / cnt - mean * mean
    s = jnp.sqrt(var + EPS)
    g = g_ref[...]
    yext = jnp.where(g >= 0.0, mx_ref[...], mn_ref[...])
    out_ref[...] = jnp.maximum((yext - mean) / s * g + b_ref[...], 0.0)


def _statsepi(mx, mn, s1, s2, gamma, beta, interpret=False):
    o = mx.shape[-1]
    return pl.pallas_call(
        _statsepi_body,
        out_shape=jax.ShapeDtypeStruct((B * N, o), jnp.float32),
        interpret=interpret,
    )(mx, mn, s1, s2, gamma.reshape(1, o), beta.reshape(1, o))


def _sb5_body(ft_ref, w_ref, g_ref, b_ref, sb_ref):
    f = ft_ref[...]
    w = w_ref[...]
    smat = lax.dot_general(f, f, (((0,), (0,)), ((), ())),
                           preferred_element_type=jnp.float32,
                           precision=lax.Precision.HIGHEST)
    mv = jnp.sum(f, axis=0, keepdims=True)
    u = lax.dot_general(w, smat, (((1,), (0,)), ((), ())),
                        preferred_element_type=jnp.float32,
                        precision=lax.Precision.HIGHEST)
    cnt = float(B * N)
    m2 = jnp.sum(u * w, axis=1, keepdims=True) / cnt
    mean = lax.dot_general(w, mv, (((1,), (1,)), ((), ())),
                           preferred_element_type=jnp.float32,
                           precision=lax.Precision.HIGHEST) / cnt
    var = m2 - mean * mean
    scale = g_ref[...] * lax.rsqrt(var + EPS)
    bias = b_ref[...] - mean * scale
    sb_ref[...] = jnp.concatenate([scale, bias], axis=1)


def _sb5(ft5, w5, g5, b5, interpret=False):
    return pl.pallas_call(
        _sb5_body,
        out_shape=jax.ShapeDtypeStruct((EMB, 2), jnp.float32),
        interpret=interpret,
    )(ft5, w5, g5.reshape(EMB, 1), b5.reshape(EMB, 1))


def _final_body(ft_ref, w_ref, sb_ref, out_ref):
    f = ft_ref[0]
    y = lax.dot_general(w_ref[...], f, (((1,), (1,)), ((), ())),
                        preferred_element_type=jnp.float32)
    scale = sb_ref[:, 0:1]
    bias = sb_ref[:, 1:2]
    out_ref[0] = jnp.maximum(y * scale + bias, 0.0)


def _final(ft5, w5, sb5, interpret=False):
    c5 = ft5.shape[-1]
    return pl.pallas_call(
        _final_body,
        grid=(B,),
        in_specs=[
            pl.BlockSpec((1, N, c5), lambda b: (b, 0, 0)),
            pl.BlockSpec((EMB, c5), lambda b: (0, 0)),
            pl.BlockSpec((EMB, 2), lambda b: (0, 0)),
        ],
        out_specs=pl.BlockSpec((1, EMB, N), lambda b: (b, 0, 0)),
        out_shape=jax.ShapeDtypeStruct((B, EMB, N), jnp.float32),
        interpret=interpret,
    )(ft5.reshape(B, N, c5), w5, sb5)


def _pad_lanes(a, cp):
    c = a.shape[-1]
    if c == cp:
        return a
    return jnp.pad(a, [(0, 0)] * (a.ndim - 1) + [(0, cp - c)])


def _xx_like_ref(ft):
    fc = jnp.transpose(ft, (0, 2, 1))
    return jnp.sum(fc * fc, axis=1, keepdims=True)


def _stage(ft_knn, ft_cat, w, gamma, beta, sc_rel, cp, interpret=False):
    idx = _knn(ft_knn, _xx_like_ref(ft_knn),
               interpret=interpret).reshape(B * N * K)
    o = w.shape[0]
    c = ft_cat.shape[-1]
    ftp = _pad_lanes(ft_cat, cp)
    w2c = jnp.concatenate([_pad_lanes(w[:, :c], cp).T,
                           _pad_lanes(w[:, c:], cp).T], axis=0)
    rel = sc_rel(ftp.reshape(B * N, cp), idx)
    mx, mn, s1, s2 = _edgeconv(rel, ftp, w2c, interpret=interpret)
    net = _statsepi(mx, mn, s1, s2, gamma, beta, interpret=interpret)
    return net.reshape(B, N, o)


def kernel(x, W1, W2, W3, W4, W5, g1, b1, g2, b2, g3, b3, g4, b4, g5, b5):
    xt = jnp.transpose(x, (0, 2, 1))

    net1 = _stage(xt, xt, W1, g1, b1, _make_sc_rel(16), 16)
    cat2 = jnp.concatenate([xt, net1], axis=-1)
    net2 = _stage(net1, cat2, W2, g2, b2, _make_sc_rel(80), 80)
    cat3 = jnp.concatenate([cat2, net2], axis=-1)
    net3 = _stage(net2, cat3, W3, g3, b3, _make_sc_rel(144), 144)
    cat4 = jnp.concatenate([cat3, net3], axis=-1)
    net4 = _stage(net3, cat4, W4, g4, b4, _make_sc_rel(208), 208)
    cat5 = jnp.concatenate([cat4, net4], axis=-1)

    ft5 = cat5.reshape(B * N, cat5.shape[-1])
    sb5 = _sb5(ft5, W5, g5, b5)
    out = _final(ft5, W5, sb5)
    return (out[:B // 2], out[B // 2:])
